# Initial kernel scaffold; baseline (speedup 1.0000x reference)
import jax
import jax.numpy as jnp
from jax.experimental import pallas as pl


def kernel(class_logits, boxes_preds, class_labels, boxes_labels):
    raise NotImplementedError("write your pallas kernel here")


# speedup vs baseline: 1.3608x; 1.3608x over previous
import functools

import jax
import jax.numpy as jnp
from jax.experimental import pallas as pl
from jax.experimental.pallas import tpu as pltpu

_FOCAL_ALPHA = 0.1
_FOCAL_GAMMA = 0.2
_NUM_CLASSES = 80
_CLASS_W, _L1_W, _GIOU_W = 2.0, 5.0, 2.0


def _loss_kernel(logits_ref, bp_ref, blT_ref, lab_ref, out_ref, *, Q, T, C):
    b = pl.program_id(0)

    @pl.when(b == 0)
    def _init():
        out_ref[0] = 0.0
        out_ref[1] = 0.0
        out_ref[2] = 0.0

    x = logits_ref[0]
    bp = bp_ref[0]
    blT = blT_ref[0]
    lab = lab_ref[0]

    p = jax.nn.sigmoid(x)
    neg = (1.0 - _FOCAL_ALPHA) * jnp.power(p, _FOCAL_GAMMA) * (
        -jnp.log(1.0 - p + 1e-8))
    pos = _FOCAL_ALPHA * jnp.power(1.0 - p, _FOCAL_GAMMA) * (
        -jnp.log(p + 1e-8))
    cm = pos - neg

    cio = jax.lax.broadcasted_iota(jnp.int32, (C, T), 0)
    oh = (cio == lab).astype(jnp.float32)
    cost_class = jax.lax.dot_general(
        cm, oh, (((1,), (0,)), ((), ())),
        preferred_element_type=jnp.float32,
        precision=jax.lax.Precision.HIGHEST)

    bx0, by0, bx1, by1 = (bp[:, i:i + 1] for i in range(4))
    lx0, ly0, lx1, ly1 = (blT[i:i + 1, :] for i in range(4))

    cost_bbox = (jnp.abs(bx0 - lx0) + jnp.abs(by0 - ly0)
                 + jnp.abs(bx1 - lx1) + jnp.abs(by1 - ly1))

    area_a = (bx1 - bx0) * (by1 - by0)
    area_b = (lx1 - lx0) * (ly1 - ly0)
    w = jnp.maximum(jnp.minimum(bx1, lx1) - jnp.maximum(bx0, lx0), 0.0)
    h = jnp.maximum(jnp.minimum(by1, ly1) - jnp.maximum(by0, ly0), 0.0)
    inter = w * h
    union = area_a + area_b - inter
    iou = inter / (union + 1e-7)
    we = jnp.maximum(jnp.maximum(bx1, lx1) - jnp.minimum(bx0, lx0), 0.0)
    he = jnp.maximum(jnp.maximum(by1, ly1) - jnp.minimum(by0, ly0), 0.0)
    area_e = we * he
    giou = iou - (area_e - union) / (area_e + 1e-7)

    cost = (_CLASS_W * cost_class + _L1_W * cost_bbox - _GIOU_W * giou)

    minv = jnp.min(cost, axis=0, keepdims=True)
    qio = jax.lax.broadcasted_iota(jnp.int32, (Q, T), 0)
    src = jnp.min(jnp.where(cost == minv, qio, Q), axis=0,
                  keepdims=True)

    m = qio == src
    mf = m.astype(jnp.float32)
    mx0 = jnp.sum(mf * bx0, axis=0, keepdims=True)
    my0 = jnp.sum(mf * by0, axis=0, keepdims=True)
    mx1 = jnp.sum(mf * bx1, axis=0, keepdims=True)
    my1 = jnp.sum(mf * by1, axis=0, keepdims=True)

    area_m = (mx1 - mx0) * (my1 - my0)
    ew = jnp.maximum(jnp.minimum(mx1, lx1) - jnp.maximum(mx0, lx0), 0.0)
    eh = jnp.maximum(jnp.minimum(my1, ly1) - jnp.maximum(my0, ly0), 0.0)
    einter = ew * eh
    eunion = area_m + area_b - einter
    eiou = einter / (eunion + 1e-7)
    ewe = jnp.maximum(jnp.maximum(mx1, lx1) - jnp.minimum(mx0, lx0), 0.0)
    ehe = jnp.maximum(jnp.maximum(my1, ly1) - jnp.minimum(my0, ly0), 0.0)
    earea = ewe * ehe
    egiou = eiou - (earea - eunion) / (earea + 1e-7)
    bbox_loc = jnp.sum(1.0 - egiou)
    bbox_l1 = jnp.sum(jnp.abs(mx0 - lx0) + jnp.abs(my0 - ly0)
                      + jnp.abs(mx1 - lx1) + jnp.abs(my1 - ly1))

    tio = jax.lax.broadcasted_iota(jnp.int32, (Q, T), 1)
    wt = jnp.max(jnp.where(m, tio, -1), axis=1, keepdims=True)
    wl = jnp.sum(jnp.where(tio == wt, jnp.broadcast_to(lab, (Q, T)), 0),
                 axis=1, keepdims=True)
    tc = jnp.where(wt >= 0, wl, C)

    lm = jax.lax.broadcasted_iota(jnp.int32, (Q, C), 1) == tc
    lf = lm.astype(jnp.float32)
    ce = (jnp.maximum(x, 0.0) - x * lf
          + jnp.log1p(jnp.exp(-jnp.abs(x))))
    p_t = jnp.where(lm, p, 1.0 - p)
    loss = ce * jnp.power(1.0 - p_t, _FOCAL_GAMMA)
    alpha_t = jnp.where(lm, _FOCAL_ALPHA, 1.0 - _FOCAL_ALPHA)
    cls_loss = jnp.sum(alpha_t * loss)

    out_ref[0] += cls_loss
    out_ref[1] += bbox_loc
    out_ref[2] += bbox_l1


def kernel(class_logits, boxes_preds, class_labels, boxes_labels):
    B, Q, C = class_logits.shape
    T = class_labels.shape[1]
    blT = boxes_labels.transpose(0, 2, 1)
    lab3 = class_labels.reshape(B, 1, T)

    out = pl.pallas_call(
        functools.partial(_loss_kernel, Q=Q, T=T, C=C),
        grid=(B,),
        in_specs=[
            pl.BlockSpec((1, Q, C), lambda b: (b, 0, 0)),
            pl.BlockSpec((1, Q, 4), lambda b: (b, 0, 0)),
            pl.BlockSpec((1, 4, T), lambda b: (b, 0, 0)),
            pl.BlockSpec((1, 1, T), lambda b: (b, 0, 0)),
        ],
        out_specs=pl.BlockSpec(memory_space=pltpu.SMEM),
        out_shape=jax.ShapeDtypeStruct((3,), jnp.float32),
    )(class_logits, boxes_preds, blT, lab3)
    return (out[0], out[1], out[2])


# speedup vs baseline: 1.6161x; 1.1876x over previous
import functools

import jax
import jax.numpy as jnp
from jax.experimental import pallas as pl
from jax.experimental.pallas import tpu as pltpu

_FOCAL_ALPHA = 0.1
_FOCAL_GAMMA = 0.2
_NUM_CLASSES = 80
_CLASS_W, _L1_W, _GIOU_W = 2.0, 5.0, 2.0


def _loss_kernel(logits_ref, bp_ref, blT_ref, lab_ref, out_ref, *, Q, T, C):
    b = pl.program_id(0)

    @pl.when(b == 0)
    def _init():
        out_ref[0] = 0.0
        out_ref[1] = 0.0
        out_ref[2] = 0.0

    x = logits_ref[0]
    bp = bp_ref[0]
    blT = blT_ref[0]
    lab = lab_ref[0]

    relu_x = jnp.maximum(x, 0.0)
    sp = jnp.log1p(jnp.exp(-jnp.abs(x)))
    logp = x - relu_x - sp
    log1mp = -relu_x - sp
    pow_p = jnp.exp(_FOCAL_GAMMA * logp)
    pow_1mp = jnp.exp(_FOCAL_GAMMA * log1mp)
    neg = (1.0 - _FOCAL_ALPHA) * pow_p * (-log1mp)
    pos = _FOCAL_ALPHA * pow_1mp * (-logp)
    cm = pos - neg

    cio = jax.lax.broadcasted_iota(jnp.int32, (C, T), 0)
    oh = (cio == lab).astype(jnp.float32)
    cost_class = jax.lax.dot_general(
        cm, oh, (((1,), (0,)), ((), ())),
        preferred_element_type=jnp.float32,
        precision=jax.lax.Precision.HIGHEST)

    bx0, by0, bx1, by1 = (bp[:, i:i + 1] for i in range(4))
    lx0, ly0, lx1, ly1 = (blT[i:i + 1, :] for i in range(4))

    cost_bbox = (jnp.abs(bx0 - lx0) + jnp.abs(by0 - ly0)
                 + jnp.abs(bx1 - lx1) + jnp.abs(by1 - ly1))

    area_a = (bx1 - bx0) * (by1 - by0)
    area_b = (lx1 - lx0) * (ly1 - ly0)
    w = jnp.maximum(jnp.minimum(bx1, lx1) - jnp.maximum(bx0, lx0), 0.0)
    h = jnp.maximum(jnp.minimum(by1, ly1) - jnp.maximum(by0, ly0), 0.0)
    inter = w * h
    union = area_a + area_b - inter
    iou = inter / (union + 1e-7)
    we = jnp.maximum(jnp.maximum(bx1, lx1) - jnp.minimum(bx0, lx0), 0.0)
    he = jnp.maximum(jnp.maximum(by1, ly1) - jnp.minimum(by0, ly0), 0.0)
    area_e = we * he
    giou = iou - (area_e - union) / (area_e + 1e-7)

    cost = (_CLASS_W * cost_class + _L1_W * cost_bbox - _GIOU_W * giou)

    minv = jnp.min(cost, axis=0, keepdims=True)
    qio = jax.lax.broadcasted_iota(jnp.int32, (Q, T), 0)
    src = jnp.min(jnp.where(cost == minv, qio, Q), axis=0,
                  keepdims=True)

    m = qio == src
    mf = m.astype(jnp.float32)
    mx0 = jnp.sum(mf * bx0, axis=0, keepdims=True)
    my0 = jnp.sum(mf * by0, axis=0, keepdims=True)
    mx1 = jnp.sum(mf * bx1, axis=0, keepdims=True)
    my1 = jnp.sum(mf * by1, axis=0, keepdims=True)

    area_m = (mx1 - mx0) * (my1 - my0)
    ew = jnp.maximum(jnp.minimum(mx1, lx1) - jnp.maximum(mx0, lx0), 0.0)
    eh = jnp.maximum(jnp.minimum(my1, ly1) - jnp.maximum(my0, ly0), 0.0)
    einter = ew * eh
    eunion = area_m + area_b - einter
    eiou = einter / (eunion + 1e-7)
    ewe = jnp.maximum(jnp.maximum(mx1, lx1) - jnp.minimum(mx0, lx0), 0.0)
    ehe = jnp.maximum(jnp.maximum(my1, ly1) - jnp.minimum(my0, ly0), 0.0)
    earea = ewe * ehe
    egiou = eiou - (earea - eunion) / (earea + 1e-7)
    bbox_loc = jnp.sum(1.0 - egiou)
    bbox_l1 = jnp.sum(jnp.abs(mx0 - lx0) + jnp.abs(my0 - ly0)
                      + jnp.abs(mx1 - lx1) + jnp.abs(my1 - ly1))

    tio = jax.lax.broadcasted_iota(jnp.int32, (Q, T), 1)
    wt = jnp.max(jnp.where(m, tio, -1), axis=1, keepdims=True)
    wl = jnp.sum(jnp.where(tio == wt, jnp.broadcast_to(lab, (Q, T)), 0),
                 axis=1, keepdims=True)
    tc = jnp.where(wt >= 0, wl, C)

    lm = jax.lax.broadcasted_iota(jnp.int32, (Q, C), 1) == tc
    ce = jnp.where(lm, relu_x - x + sp, relu_x + sp)
    powterm = jnp.where(lm, pow_1mp, pow_p)
    alpha_t = jnp.where(lm, _FOCAL_ALPHA, 1.0 - _FOCAL_ALPHA)
    cls_loss = jnp.sum(alpha_t * ce * powterm)

    out_ref[0] += cls_loss
    out_ref[1] += bbox_loc
    out_ref[2] += bbox_l1


def kernel(class_logits, boxes_preds, class_labels, boxes_labels):
    B, Q, C = class_logits.shape
    T = class_labels.shape[1]
    blT = boxes_labels.transpose(0, 2, 1)
    lab3 = class_labels.reshape(B, 1, T)

    out = pl.pallas_call(
        functools.partial(_loss_kernel, Q=Q, T=T, C=C),
        grid=(B,),
        in_specs=[
            pl.BlockSpec((1, Q, C), lambda b: (b, 0, 0)),
            pl.BlockSpec((1, Q, 4), lambda b: (b, 0, 0)),
            pl.BlockSpec((1, 4, T), lambda b: (b, 0, 0)),
            pl.BlockSpec((1, 1, T), lambda b: (b, 0, 0)),
        ],
        out_specs=pl.BlockSpec(memory_space=pltpu.SMEM),
        out_shape=jax.ShapeDtypeStruct((3,), jnp.float32),
    )(class_logits, boxes_preds, blT, lab3)
    return (out[0], out[1], out[2])
